# baseline (speedup 1.0000x reference)
import jax
import jax.numpy as jnp
from jax import lax
from jax.experimental import pallas as pl
from jax.experimental.pallas import tpu as pltpu
from jax.experimental.pallas import tpu_sc as plsc

IN_F = 16384
OUT_F = 1024
NNZ = 16777

L = 16
NW = 32
ROWS_PER_W = IN_F // NW
SB_ROWS = 16
NUM_SB = ROWS_PER_W // SB_ROWS
CH = 1536
NNZ_PAD = 18432
N_CHUNKS = NNZ_PAD // CH
COMP_CAP = 16816
WORK_CAP = CH
SENTINEL = 2 ** 30


def _splat(x):
    return jnp.full((L,), x, dtype=jnp.int32)


def _scalar(v):
    return jnp.max(v)


def _sc_body(x_hbm, rows_hbm, cols_hbm, vals_hbm, bias_hbm, out_hbm,
             s0r, s0c, s0v, s1r, s1c, s1v, comp_off, comp_col, comp_val,
             acc_a, acc_b, xbuf_a, xbuf_b, bias_v,
             sem_a, sem_b, osem_a, osem_b):
    cid = lax.axis_index("c")
    sid = lax.axis_index("s")
    wid = sid * 2 + cid
    base = wid * ROWS_PER_W
    base_v = _splat(base)
    iota = lax.iota(jnp.int32, L)
    zero_v = _splat(0)

    pltpu.sync_copy(bias_hbm, bias_v)

    def chunk_issue(c, st):
        pltpu.async_copy(rows_hbm.at[pl.ds(c * CH, CH)], st[0], st[3])
        pltpu.async_copy(cols_hbm.at[pl.ds(c * CH, CH)], st[1], st[3])
        pltpu.async_copy(vals_hbm.at[pl.ds(c * CH, CH)], st[2], st[3])

    def chunk_wait(c, st):
        pltpu.make_async_copy(rows_hbm.at[pl.ds(c * CH, CH)], st[0], st[3]).wait()
        pltpu.make_async_copy(cols_hbm.at[pl.ds(c * CH, CH)], st[1], st[3]).wait()
        pltpu.make_async_copy(vals_hbm.at[pl.ds(c * CH, CH)], st[2], st[3]).wait()

    def scan_set(st, m_v):
        cr, cc_ref, cv = st[0], st[1], st[2]

        @plsc.parallel_loop(0, CH // L, carry=m_v, unroll=4)
        def _scan(g, mv):
            r = cr[pl.ds(g * L, L)]
            cc = cc_ref[pl.ds(g * L, L)]
            vv = cv[pl.ds(g * L, L)]
            off = r - base_v
            mask = (r >= base_v) & (off < _splat(ROWS_PER_W))
            cnt_v = plsc.all_reduce_population_count(mask)
            pref = plsc.cumsum(mask.astype(jnp.int32))
            pos = mv + pref - 1
            plsc.store_scatter(comp_off, [pos], off, mask=mask)
            plsc.store_scatter(comp_col, [pos], cc, mask=mask)
            plsc.store_scatter(comp_val, [pos], vv, mask=mask)
            return mv + cnt_v

        return _scan

    set_a = (s0r, s0c, s0v, sem_a)
    set_b = (s1r, s1c, s1v, sem_b)

    chunk_issue(0, set_a)

    def scan_pair(t, m_v):
        ca = 2 * t
        chunk_wait(ca, set_a)
        chunk_issue(ca + 1, set_b)
        m_v = scan_set(set_a, m_v)
        chunk_wait(ca + 1, set_b)

        @pl.when(ca + 2 < N_CHUNKS)
        def _next():
            chunk_issue(ca + 2, set_a)

        return scan_set(set_b, m_v)

    m_v = lax.fori_loop(0, N_CHUNKS // 2, scan_pair, zero_v)
    m_total = _scalar(m_v)
    m_vv = _splat(m_total)
    n_comp_groups = (m_total + (L - 1)) // L

    def fill_acc(accx):
        @plsc.parallel_loop(0, OUT_F // L, unroll=4)
        def _fill(k):
            bv = bias_v[pl.ds(k * L, L)]
            for r in range(SB_ROWS):
                accx[r, pl.ds(k * L, L)] = bv

        return None

    def gather_issue(g, stc, xbufx, semx):
        pltpu.async_copy(x_hbm.at[stc.at[pl.ds(g * L, L)]], xbufx, semx)

    def gather_wait(g, stc, xbufx, semx):
        pltpu.make_async_copy(
            x_hbm.at[stc.at[pl.ds(g * L, L)]], xbufx, semx).wait()

    def one_group(g, n_w, str_, stv, xbufx, accx):
        n_here = jnp.minimum(n_w - g * L, L)
        g_l = g * L

        @plsc.parallel_loop(0, n_here)
        def _entry(j):
            ev = _splat(g_l + j)
            row_v = plsc.load_gather(str_, [ev])
            val_v = plsc.load_gather(stv, [ev])
            for k in range(OUT_F // L):
                c0 = k * L
                xv = xbufx[j, pl.ds(c0, L)]
                plsc.addupdate_scatter(accx, [row_v, iota + c0], val_v * xv)

        return None

    def process_block(n_w, st, xbufx, accx):
        str_, stc, stv, semx = st
        ng = (n_w + (L - 1)) // L

        def _g(g, _):
            gather_wait(g, stc, xbufx, semx)
            one_group(g, n_w, str_, stv, xbufx, accx)

            @pl.when(g + 1 < ng)
            def _pre():
                gather_issue(g + 1, stc, xbufx, semx)

            return 0

        lax.fori_loop(0, ng, _g, 0)

    def subscan_fast(lo, st):
        str_, stc, stv = st[0], st[1], st[2]
        lo_v = _splat(lo)

        @plsc.parallel_loop(0, n_comp_groups, carry=zero_v, unroll=4)
        def _ss(g, nv):
            off = comp_off[pl.ds(g * L, L)]
            cc = comp_col[pl.ds(g * L, L)]
            vv = comp_val[pl.ds(g * L, L)]
            in_blk = off - lo_v
            mask = ((off >= lo_v) & (in_blk < _splat(SB_ROWS))
                    & ((_splat(g * L) + iota) < m_vv))
            cnt_v = plsc.all_reduce_population_count(mask)
            pref = plsc.cumsum(mask.astype(jnp.int32))
            pos = nv + pref - 1
            plsc.store_scatter(str_, [pos], in_blk, mask=mask)
            plsc.store_scatter(stc, [pos], cc, mask=mask)
            plsc.store_scatter(stv, [pos], vv, mask=mask)
            return nv + cnt_v

        return _scalar(_ss)

    def out_rows(b):
        return out_hbm.at[pl.ds(base + b * SB_ROWS, SB_ROWS)]

    @pl.when(m_total <= WORK_CAP)
    def _fast():
        n0 = subscan_fast(0, set_a)

        @pl.when(n0 > 0)
        def _p0():
            gather_issue(0, set_a[1], xbuf_a, sem_a)

        def pair(t, n_cur):
            b_a = 2 * t

            @pl.when(t > 0)
            def _wa():
                pltpu.make_async_copy(acc_a, out_rows(b_a - 2), osem_a).wait()

            fill_acc(acc_a)
            n_b = subscan_fast((b_a + 1) * SB_ROWS, set_b)

            @pl.when(n_b > 0)
            def _pb():
                gather_issue(0, set_b[1], xbuf_b, sem_b)

            process_block(n_cur, set_a, xbuf_a, acc_a)
            pltpu.async_copy(acc_a, out_rows(b_a), osem_a)

            @pl.when(t > 0)
            def _wb():
                pltpu.make_async_copy(acc_b, out_rows(b_a - 1), osem_b).wait()

            fill_acc(acc_b)
            n_next = subscan_fast((b_a + 2) * SB_ROWS, set_a)

            @pl.when(n_next > 0)
            def _pn():
                gather_issue(0, set_a[1], xbuf_a, sem_a)

            process_block(n_b, set_b, xbuf_b, acc_b)
            pltpu.async_copy(acc_b, out_rows(b_a + 1), osem_b)
            return n_next

        lax.fori_loop(0, NUM_SB // 2, pair, n0)
        pltpu.make_async_copy(acc_a, out_rows(NUM_SB - 2), osem_a).wait()
        pltpu.make_async_copy(acc_b, out_rows(NUM_SB - 1), osem_b).wait()

    @pl.when(m_total > WORK_CAP)
    def _slow():
        def do_block(sb, _):
            lo = sb * SB_ROWS
            lo_v = _splat(lo)
            fill_acc(acc_a)

            def process_now(n_w):
                ng = (n_w + (L - 1)) // L

                def _g(g, _):
                    gather_issue(g, set_a[1], xbuf_a, sem_a)
                    gather_wait(g, set_a[1], xbuf_a, sem_a)
                    one_group(g, n_w, set_a[0], set_a[2], xbuf_a, acc_a)
                    return 0

                lax.fori_loop(0, ng, _g, 0)

            def sub_scan(g, n_w):
                off = comp_off[pl.ds(g * L, L)]
                cc = comp_col[pl.ds(g * L, L)]
                vv = comp_val[pl.ds(g * L, L)]
                in_blk = off - lo_v
                mask = ((off >= lo_v) & (in_blk < _splat(SB_ROWS))
                        & ((_splat(g * L) + iota) < m_vv))

                @pl.when(n_w + L > WORK_CAP)
                def _flush():
                    process_now(n_w)

                n_w = jnp.where(n_w + L > WORK_CAP, 0, n_w)
                pref = plsc.cumsum(mask.astype(jnp.int32))
                pos = _splat(n_w) + pref - 1
                plsc.store_scatter(set_a[0], [pos], in_blk, mask=mask)
                plsc.store_scatter(set_a[1], [pos], cc, mask=mask)
                plsc.store_scatter(set_a[2], [pos], vv, mask=mask)
                return n_w + _scalar(pref)

            n_w = lax.fori_loop(0, n_comp_groups, sub_scan, jnp.int32(0))
            process_now(n_w)
            pltpu.sync_copy(acc_a, out_rows(sb))
            return 0

        lax.fori_loop(0, NUM_SB, do_block, 0)


@jax.jit
def kernel(_x, w_indices, w_values, bias):
    rows = jnp.full((NNZ_PAD,), SENTINEL, jnp.int32).at[:NNZ].set(w_indices[0])
    cols = jnp.zeros((NNZ_PAD,), jnp.int32).at[:NNZ].set(w_indices[1])
    vals = jnp.zeros((NNZ_PAD,), jnp.float32).at[:NNZ].set(w_values)

    mesh = plsc.VectorSubcoreMesh(core_axis_name="c", subcore_axis_name="s",
                                  num_cores=2, num_subcores=16)
    run = pl.kernel(
        _sc_body,
        out_type=jax.ShapeDtypeStruct((IN_F, OUT_F), jnp.float32),
        mesh=mesh,
        compiler_params=pltpu.CompilerParams(needs_layout_passes=False),
        scratch_types=[
            pltpu.VMEM((CH,), jnp.int32),
            pltpu.VMEM((CH,), jnp.int32),
            pltpu.VMEM((CH,), jnp.float32),
            pltpu.VMEM((CH,), jnp.int32),
            pltpu.VMEM((CH,), jnp.int32),
            pltpu.VMEM((CH,), jnp.float32),
            pltpu.VMEM((COMP_CAP,), jnp.int32),
            pltpu.VMEM((COMP_CAP,), jnp.int32),
            pltpu.VMEM((COMP_CAP,), jnp.float32),
            pltpu.VMEM((SB_ROWS, OUT_F), jnp.float32),
            pltpu.VMEM((SB_ROWS, OUT_F), jnp.float32),
            pltpu.VMEM((L, OUT_F), jnp.float32),
            pltpu.VMEM((L, OUT_F), jnp.float32),
            pltpu.VMEM((OUT_F,), jnp.float32),
            pltpu.SemaphoreType.DMA,
            pltpu.SemaphoreType.DMA,
            pltpu.SemaphoreType.DMA,
            pltpu.SemaphoreType.DMA,
        ],
    )
    return run(_x, rows, cols, vals, bias)


# speedup vs baseline: 1.0256x; 1.0256x over previous
import jax
import jax.numpy as jnp
from jax import lax
from jax.experimental import pallas as pl
from jax.experimental.pallas import tpu as pltpu
from jax.experimental.pallas import tpu_sc as plsc

IN_F = 16384
OUT_F = 1024
NNZ = 16777

L = 16
NW = 32
ROWS_PER_W = IN_F // NW
SB_ROWS = 16
NUM_SB = ROWS_PER_W // SB_ROWS
CH = 1536
NNZ_PAD = 18432
N_CHUNKS = NNZ_PAD // CH
COMP_CAP = 16816
WORK_CAP = CH
SENTINEL = 2 ** 30


def _splat(x):
    return jnp.full((L,), x, dtype=jnp.int32)


def _scalar(v):
    return jnp.max(v)


def _sc_body(x_hbm, rows_hbm, cols_hbm, vals_hbm, bias_hbm, out_hbm,
             s0r, s0c, s0v, s1r, s1c, s1v, comp_off, comp_col, comp_val,
             acc_a, acc_b, xbuf_a, xbuf_b, bias_v,
             sem_a, sem_b, osem_a, osem_b):
    cid = lax.axis_index("c")
    sid = lax.axis_index("s")
    wid = sid * 2 + cid
    base = wid * ROWS_PER_W
    base_v = _splat(base)
    iota = lax.iota(jnp.int32, L)
    zero_v = _splat(0)

    pltpu.sync_copy(bias_hbm, bias_v)

    def chunk_issue(c, st):
        pltpu.async_copy(rows_hbm.at[pl.ds(c * CH, CH)], st[0], st[3])
        pltpu.async_copy(cols_hbm.at[pl.ds(c * CH, CH)], st[1], st[3])
        pltpu.async_copy(vals_hbm.at[pl.ds(c * CH, CH)], st[2], st[3])

    def chunk_wait(c, st):
        pltpu.make_async_copy(rows_hbm.at[pl.ds(c * CH, CH)], st[0], st[3]).wait()
        pltpu.make_async_copy(cols_hbm.at[pl.ds(c * CH, CH)], st[1], st[3]).wait()
        pltpu.make_async_copy(vals_hbm.at[pl.ds(c * CH, CH)], st[2], st[3]).wait()

    def scan_set(st, m_v):
        cr, cc_ref, cv = st[0], st[1], st[2]

        @plsc.parallel_loop(0, CH // L, carry=m_v)
        def _scan(g, mv):
            r = cr[pl.ds(g * L, L)]
            cc = cc_ref[pl.ds(g * L, L)]
            vv = cv[pl.ds(g * L, L)]
            off = r - base_v
            mask = (r >= base_v) & (off < _splat(ROWS_PER_W))
            cnt_v = plsc.all_reduce_population_count(mask)
            pref = plsc.cumsum(mask.astype(jnp.int32))
            pos = mv + pref - 1
            plsc.store_scatter(comp_off, [pos], off, mask=mask)
            plsc.store_scatter(comp_col, [pos], cc, mask=mask)
            plsc.store_scatter(comp_val, [pos], vv, mask=mask)
            return mv + cnt_v

        return _scan

    set_a = (s0r, s0c, s0v, sem_a)
    set_b = (s1r, s1c, s1v, sem_b)

    chunk_issue(0, set_a)

    def scan_pair(t, m_v):
        ca = 2 * t
        chunk_wait(ca, set_a)
        chunk_issue(ca + 1, set_b)
        m_v = scan_set(set_a, m_v)
        chunk_wait(ca + 1, set_b)

        @pl.when(ca + 2 < N_CHUNKS)
        def _next():
            chunk_issue(ca + 2, set_a)

        return scan_set(set_b, m_v)

    m_v = lax.fori_loop(0, N_CHUNKS // 2, scan_pair, zero_v)
    m_total = _scalar(m_v)
    m_vv = _splat(m_total)
    n_comp_groups = (m_total + (L - 1)) // L

    def fill_acc(accx):
        @plsc.parallel_loop(0, OUT_F // L)
        def _fill(k):
            bv = bias_v[pl.ds(k * L, L)]
            for r in range(SB_ROWS):
                accx[r, pl.ds(k * L, L)] = bv

        return None

    def gather_issue(g, stc, xbufx, semx):
        pltpu.async_copy(x_hbm.at[stc.at[pl.ds(g * L, L)]], xbufx, semx)

    def gather_wait(g, stc, xbufx, semx):
        pltpu.make_async_copy(
            x_hbm.at[stc.at[pl.ds(g * L, L)]], xbufx, semx).wait()

    def one_group(g, n_w, str_, stv, xbufx, accx):
        n_here = jnp.minimum(n_w - g * L, L)
        g_l = g * L

        @plsc.parallel_loop(0, n_here)
        def _entry(j):
            ev = _splat(g_l + j)
            row_v = plsc.load_gather(str_, [ev])
            val_v = plsc.load_gather(stv, [ev])
            for k in range(OUT_F // L):
                c0 = k * L
                xv = xbufx[j, pl.ds(c0, L)]
                plsc.addupdate_scatter(accx, [row_v, iota + c0], val_v * xv)

        return None

    def process_block(n_w, st, xbufx, accx):
        str_, stc, stv, semx = st
        ng = (n_w + (L - 1)) // L

        def _g(g, _):
            gather_wait(g, stc, xbufx, semx)
            one_group(g, n_w, str_, stv, xbufx, accx)

            @pl.when(g + 1 < ng)
            def _pre():
                gather_issue(g + 1, stc, xbufx, semx)

            return 0

        lax.fori_loop(0, ng, _g, 0)

    def subscan_fast(lo, st):
        str_, stc, stv = st[0], st[1], st[2]
        lo_v = _splat(lo)

        @plsc.parallel_loop(0, n_comp_groups, carry=zero_v)
        def _ss(g, nv):
            off = comp_off[pl.ds(g * L, L)]
            cc = comp_col[pl.ds(g * L, L)]
            vv = comp_val[pl.ds(g * L, L)]
            in_blk = off - lo_v
            mask = ((off >= lo_v) & (in_blk < _splat(SB_ROWS))
                    & ((_splat(g * L) + iota) < m_vv))
            cnt_v = plsc.all_reduce_population_count(mask)
            pref = plsc.cumsum(mask.astype(jnp.int32))
            pos = nv + pref - 1
            plsc.store_scatter(str_, [pos], in_blk, mask=mask)
            plsc.store_scatter(stc, [pos], cc, mask=mask)
            plsc.store_scatter(stv, [pos], vv, mask=mask)
            return nv + cnt_v

        return _scalar(_ss)

    def out_rows(b):
        return out_hbm.at[pl.ds(base + b * SB_ROWS, SB_ROWS)]

    @pl.when(m_total <= WORK_CAP)
    def _fast():
        n0 = subscan_fast(0, set_a)

        @pl.when(n0 > 0)
        def _p0():
            gather_issue(0, set_a[1], xbuf_a, sem_a)

        def pair(t, n_cur):
            b_a = 2 * t

            @pl.when(t > 0)
            def _wa():
                pltpu.make_async_copy(acc_a, out_rows(b_a - 2), osem_a).wait()

            fill_acc(acc_a)
            n_b = subscan_fast((b_a + 1) * SB_ROWS, set_b)

            @pl.when(n_b > 0)
            def _pb():
                gather_issue(0, set_b[1], xbuf_b, sem_b)

            process_block(n_cur, set_a, xbuf_a, acc_a)
            pltpu.async_copy(acc_a, out_rows(b_a), osem_a)

            @pl.when(t > 0)
            def _wb():
                pltpu.make_async_copy(acc_b, out_rows(b_a - 1), osem_b).wait()

            fill_acc(acc_b)
            n_next = subscan_fast((b_a + 2) * SB_ROWS, set_a)

            @pl.when(n_next > 0)
            def _pn():
                gather_issue(0, set_a[1], xbuf_a, sem_a)

            process_block(n_b, set_b, xbuf_b, acc_b)
            pltpu.async_copy(acc_b, out_rows(b_a + 1), osem_b)
            return n_next

        lax.fori_loop(0, NUM_SB // 2, pair, n0)
        pltpu.make_async_copy(acc_a, out_rows(NUM_SB - 2), osem_a).wait()
        pltpu.make_async_copy(acc_b, out_rows(NUM_SB - 1), osem_b).wait()

    @pl.when(m_total > WORK_CAP)
    def _slow():
        def do_block(sb, _):
            lo = sb * SB_ROWS
            lo_v = _splat(lo)
            fill_acc(acc_a)

            def process_now(n_w):
                ng = (n_w + (L - 1)) // L

                def _g(g, _):
                    gather_issue(g, set_a[1], xbuf_a, sem_a)
                    gather_wait(g, set_a[1], xbuf_a, sem_a)
                    one_group(g, n_w, set_a[0], set_a[2], xbuf_a, acc_a)
                    return 0

                lax.fori_loop(0, ng, _g, 0)

            def sub_scan(g, n_w):
                off = comp_off[pl.ds(g * L, L)]
                cc = comp_col[pl.ds(g * L, L)]
                vv = comp_val[pl.ds(g * L, L)]
                in_blk = off - lo_v
                mask = ((off >= lo_v) & (in_blk < _splat(SB_ROWS))
                        & ((_splat(g * L) + iota) < m_vv))

                @pl.when(n_w + L > WORK_CAP)
                def _flush():
                    process_now(n_w)

                n_w = jnp.where(n_w + L > WORK_CAP, 0, n_w)
                pref = plsc.cumsum(mask.astype(jnp.int32))
                pos = _splat(n_w) + pref - 1
                plsc.store_scatter(set_a[0], [pos], in_blk, mask=mask)
                plsc.store_scatter(set_a[1], [pos], cc, mask=mask)
                plsc.store_scatter(set_a[2], [pos], vv, mask=mask)
                return n_w + _scalar(pref)

            n_w = lax.fori_loop(0, n_comp_groups, sub_scan, jnp.int32(0))
            process_now(n_w)
            pltpu.sync_copy(acc_a, out_rows(sb))
            return 0

        lax.fori_loop(0, NUM_SB, do_block, 0)


@jax.jit
def kernel(_x, w_indices, w_values, bias):
    rows = jnp.full((NNZ_PAD,), SENTINEL, jnp.int32).at[:NNZ].set(w_indices[0])
    cols = jnp.zeros((NNZ_PAD,), jnp.int32).at[:NNZ].set(w_indices[1])
    vals = jnp.zeros((NNZ_PAD,), jnp.float32).at[:NNZ].set(w_values)

    mesh = plsc.VectorSubcoreMesh(core_axis_name="c", subcore_axis_name="s",
                                  num_cores=2, num_subcores=16)
    run = pl.kernel(
        _sc_body,
        out_type=jax.ShapeDtypeStruct((IN_F, OUT_F), jnp.float32),
        mesh=mesh,
        compiler_params=pltpu.CompilerParams(needs_layout_passes=False),
        scratch_types=[
            pltpu.VMEM((CH,), jnp.int32),
            pltpu.VMEM((CH,), jnp.int32),
            pltpu.VMEM((CH,), jnp.float32),
            pltpu.VMEM((CH,), jnp.int32),
            pltpu.VMEM((CH,), jnp.int32),
            pltpu.VMEM((CH,), jnp.float32),
            pltpu.VMEM((COMP_CAP,), jnp.int32),
            pltpu.VMEM((COMP_CAP,), jnp.int32),
            pltpu.VMEM((COMP_CAP,), jnp.float32),
            pltpu.VMEM((SB_ROWS, OUT_F), jnp.float32),
            pltpu.VMEM((SB_ROWS, OUT_F), jnp.float32),
            pltpu.VMEM((L, OUT_F), jnp.float32),
            pltpu.VMEM((L, OUT_F), jnp.float32),
            pltpu.VMEM((OUT_F,), jnp.float32),
            pltpu.SemaphoreType.DMA,
            pltpu.SemaphoreType.DMA,
            pltpu.SemaphoreType.DMA,
            pltpu.SemaphoreType.DMA,
        ],
    )
    return run(_x, rows, cols, vals, bias)


# speedup vs baseline: 1.0265x; 1.0009x over previous
import jax
import jax.numpy as jnp
from jax import lax
from jax.experimental import pallas as pl
from jax.experimental.pallas import tpu as pltpu
from jax.experimental.pallas import tpu_sc as plsc

IN_F = 16384
OUT_F = 1024
NNZ = 16777

L = 16
NW = 32
ROWS_PER_W = IN_F // NW
SB_ROWS = 16
NUM_SB = ROWS_PER_W // SB_ROWS
CH = 1536
NNZ_PAD = 18432
N_CHUNKS = NNZ_PAD // CH
COMP_CAP = 16816
WORK_CAP = CH
SENTINEL = 2 ** 30


def _splat(x):
    return jnp.full((L,), x, dtype=jnp.int32)


def _scalar(v):
    return jnp.max(v)


def _sc_body(x_hbm, rows_hbm, cols_hbm, vals_hbm, bias_hbm, out_hbm,
             s0r, s0c, s0v, s1r, s1c, s1v, comp_off, comp_col, comp_val,
             acc_a, acc_b, xbuf_a, xbuf_b, bias_v,
             sem_a, sem_b, osem_a, osem_b):
    cid = lax.axis_index("c")
    sid = lax.axis_index("s")
    wid = sid * 2 + cid
    base = wid * ROWS_PER_W
    base_v = _splat(base)
    iota = lax.iota(jnp.int32, L)
    zero_v = _splat(0)

    pltpu.sync_copy(bias_hbm, bias_v)

    def chunk_issue(c, st):
        pltpu.async_copy(rows_hbm.at[pl.ds(c * CH, CH)], st[0], st[3])
        pltpu.async_copy(cols_hbm.at[pl.ds(c * CH, CH)], st[1], st[3])
        pltpu.async_copy(vals_hbm.at[pl.ds(c * CH, CH)], st[2], st[3])

    def chunk_wait(c, st):
        pltpu.make_async_copy(rows_hbm.at[pl.ds(c * CH, CH)], st[0], st[3]).wait()
        pltpu.make_async_copy(cols_hbm.at[pl.ds(c * CH, CH)], st[1], st[3]).wait()
        pltpu.make_async_copy(vals_hbm.at[pl.ds(c * CH, CH)], st[2], st[3]).wait()

    def scan_set(st, m_v):
        cr, cc_ref, cv = st[0], st[1], st[2]

        @plsc.parallel_loop(0, CH // L, carry=m_v)
        def _scan(g, mv):
            r = cr[pl.ds(g * L, L)]
            cc = cc_ref[pl.ds(g * L, L)]
            vv = cv[pl.ds(g * L, L)]
            off = r - base_v
            mask = (r >= base_v) & (off < _splat(ROWS_PER_W))
            cnt_v = plsc.all_reduce_population_count(mask)
            pref = plsc.cumsum(mask.astype(jnp.int32))
            pos = mv + pref - 1
            plsc.store_scatter(comp_off, [pos], off, mask=mask)
            plsc.store_scatter(comp_col, [pos], cc, mask=mask)
            plsc.store_scatter(comp_val, [pos], vv, mask=mask)
            return mv + cnt_v

        return _scan

    set_a = (s0r, s0c, s0v, sem_a)
    set_b = (s1r, s1c, s1v, sem_b)

    chunk_issue(0, set_a)

    def scan_pair(t, m_v):
        ca = 2 * t
        chunk_wait(ca, set_a)
        chunk_issue(ca + 1, set_b)
        m_v = scan_set(set_a, m_v)
        chunk_wait(ca + 1, set_b)

        @pl.when(ca + 2 < N_CHUNKS)
        def _next():
            chunk_issue(ca + 2, set_a)

        return scan_set(set_b, m_v)

    m_v = lax.fori_loop(0, N_CHUNKS // 2, scan_pair, zero_v)
    m_total = _scalar(m_v)
    m_vv = _splat(m_total)
    n_comp_groups = (m_total + (L - 1)) // L

    def fill_acc(accx):
        @plsc.parallel_loop(0, OUT_F // L)
        def _fill(k):
            bv = bias_v[pl.ds(k * L, L)]
            for r in range(SB_ROWS):
                accx[r, pl.ds(k * L, L)] = bv

        return None

    def gather_issue(g, stc, xbufx, semx):
        pltpu.async_copy(x_hbm.at[stc.at[pl.ds(g * L, L)]], xbufx, semx)

    def gather_wait(g, stc, xbufx, semx):
        pltpu.make_async_copy(
            x_hbm.at[stc.at[pl.ds(g * L, L)]], xbufx, semx).wait()

    def one_group(g, n_w, str_, stv, xbufx, accx):
        n_here = jnp.minimum(n_w - g * L, L)
        g_l = g * L

        @plsc.parallel_loop(0, n_here)
        def _entry(j):
            ev = _splat(g_l + j)
            row_v = plsc.load_gather(str_, [ev])
            val_v = plsc.load_gather(stv, [ev])
            cv = iota
            step = _splat(L)
            for k in range(OUT_F // L):
                xv = xbufx[j, pl.ds(k * L, L)]
                plsc.addupdate_scatter(accx, [row_v, cv], val_v * xv)
                cv = cv + step

        return None

    def process_block(n_w, st, xbufx, accx):
        str_, stc, stv, semx = st
        ng = (n_w + (L - 1)) // L

        def _g(g, _):
            gather_wait(g, stc, xbufx, semx)
            one_group(g, n_w, str_, stv, xbufx, accx)

            @pl.when(g + 1 < ng)
            def _pre():
                gather_issue(g + 1, stc, xbufx, semx)

            return 0

        lax.fori_loop(0, ng, _g, 0)

    def subscan_fast(lo, st):
        str_, stc, stv = st[0], st[1], st[2]
        lo_v = _splat(lo)

        @plsc.parallel_loop(0, n_comp_groups, carry=zero_v)
        def _ss(g, nv):
            off = comp_off[pl.ds(g * L, L)]
            cc = comp_col[pl.ds(g * L, L)]
            vv = comp_val[pl.ds(g * L, L)]
            in_blk = off - lo_v
            mask = ((off >= lo_v) & (in_blk < _splat(SB_ROWS))
                    & ((_splat(g * L) + iota) < m_vv))
            cnt_v = plsc.all_reduce_population_count(mask)
            pref = plsc.cumsum(mask.astype(jnp.int32))
            pos = nv + pref - 1
            plsc.store_scatter(str_, [pos], in_blk, mask=mask)
            plsc.store_scatter(stc, [pos], cc, mask=mask)
            plsc.store_scatter(stv, [pos], vv, mask=mask)
            return nv + cnt_v

        return _scalar(_ss)

    def out_rows(b):
        return out_hbm.at[pl.ds(base + b * SB_ROWS, SB_ROWS)]

    @pl.when(m_total <= WORK_CAP)
    def _fast():
        n0 = subscan_fast(0, set_a)

        @pl.when(n0 > 0)
        def _p0():
            gather_issue(0, set_a[1], xbuf_a, sem_a)

        def pair(t, n_cur):
            b_a = 2 * t

            @pl.when(t > 0)
            def _wa():
                pltpu.make_async_copy(acc_a, out_rows(b_a - 2), osem_a).wait()

            fill_acc(acc_a)
            n_b = subscan_fast((b_a + 1) * SB_ROWS, set_b)

            @pl.when(n_b > 0)
            def _pb():
                gather_issue(0, set_b[1], xbuf_b, sem_b)

            process_block(n_cur, set_a, xbuf_a, acc_a)
            pltpu.async_copy(acc_a, out_rows(b_a), osem_a)

            @pl.when(t > 0)
            def _wb():
                pltpu.make_async_copy(acc_b, out_rows(b_a - 1), osem_b).wait()

            fill_acc(acc_b)
            n_next = subscan_fast((b_a + 2) * SB_ROWS, set_a)

            @pl.when(n_next > 0)
            def _pn():
                gather_issue(0, set_a[1], xbuf_a, sem_a)

            process_block(n_b, set_b, xbuf_b, acc_b)
            pltpu.async_copy(acc_b, out_rows(b_a + 1), osem_b)
            return n_next

        lax.fori_loop(0, NUM_SB // 2, pair, n0)
        pltpu.make_async_copy(acc_a, out_rows(NUM_SB - 2), osem_a).wait()
        pltpu.make_async_copy(acc_b, out_rows(NUM_SB - 1), osem_b).wait()

    @pl.when(m_total > WORK_CAP)
    def _slow():
        def do_block(sb, _):
            lo = sb * SB_ROWS
            lo_v = _splat(lo)
            fill_acc(acc_a)

            def process_now(n_w):
                ng = (n_w + (L - 1)) // L

                def _g(g, _):
                    gather_issue(g, set_a[1], xbuf_a, sem_a)
                    gather_wait(g, set_a[1], xbuf_a, sem_a)
                    one_group(g, n_w, set_a[0], set_a[2], xbuf_a, acc_a)
                    return 0

                lax.fori_loop(0, ng, _g, 0)

            def sub_scan(g, n_w):
                off = comp_off[pl.ds(g * L, L)]
                cc = comp_col[pl.ds(g * L, L)]
                vv = comp_val[pl.ds(g * L, L)]
                in_blk = off - lo_v
                mask = ((off >= lo_v) & (in_blk < _splat(SB_ROWS))
                        & ((_splat(g * L) + iota) < m_vv))

                @pl.when(n_w + L > WORK_CAP)
                def _flush():
                    process_now(n_w)

                n_w = jnp.where(n_w + L > WORK_CAP, 0, n_w)
                pref = plsc.cumsum(mask.astype(jnp.int32))
                pos = _splat(n_w) + pref - 1
                plsc.store_scatter(set_a[0], [pos], in_blk, mask=mask)
                plsc.store_scatter(set_a[1], [pos], cc, mask=mask)
                plsc.store_scatter(set_a[2], [pos], vv, mask=mask)
                return n_w + _scalar(pref)

            n_w = lax.fori_loop(0, n_comp_groups, sub_scan, jnp.int32(0))
            process_now(n_w)
            pltpu.sync_copy(acc_a, out_rows(sb))
            return 0

        lax.fori_loop(0, NUM_SB, do_block, 0)


@jax.jit
def kernel(_x, w_indices, w_values, bias):
    rows = jnp.full((NNZ_PAD,), SENTINEL, jnp.int32).at[:NNZ].set(w_indices[0])
    cols = jnp.zeros((NNZ_PAD,), jnp.int32).at[:NNZ].set(w_indices[1])
    vals = jnp.zeros((NNZ_PAD,), jnp.float32).at[:NNZ].set(w_values)

    mesh = plsc.VectorSubcoreMesh(core_axis_name="c", subcore_axis_name="s",
                                  num_cores=2, num_subcores=16)
    run = pl.kernel(
        _sc_body,
        out_type=jax.ShapeDtypeStruct((IN_F, OUT_F), jnp.float32),
        mesh=mesh,
        compiler_params=pltpu.CompilerParams(needs_layout_passes=False),
        scratch_types=[
            pltpu.VMEM((CH,), jnp.int32),
            pltpu.VMEM((CH,), jnp.int32),
            pltpu.VMEM((CH,), jnp.float32),
            pltpu.VMEM((CH,), jnp.int32),
            pltpu.VMEM((CH,), jnp.int32),
            pltpu.VMEM((CH,), jnp.float32),
            pltpu.VMEM((COMP_CAP,), jnp.int32),
            pltpu.VMEM((COMP_CAP,), jnp.int32),
            pltpu.VMEM((COMP_CAP,), jnp.float32),
            pltpu.VMEM((SB_ROWS, OUT_F), jnp.float32),
            pltpu.VMEM((SB_ROWS, OUT_F), jnp.float32),
            pltpu.VMEM((L, OUT_F), jnp.float32),
            pltpu.VMEM((L, OUT_F), jnp.float32),
            pltpu.VMEM((OUT_F,), jnp.float32),
            pltpu.SemaphoreType.DMA,
            pltpu.SemaphoreType.DMA,
            pltpu.SemaphoreType.DMA,
            pltpu.SemaphoreType.DMA,
        ],
    )
    return run(_x, rows, cols, vals, bias)
